# baseline (speedup 1.0000x reference)
import functools
import math

import jax
import jax.numpy as jnp
from jax.experimental import pallas as pl
from jax.experimental.pallas import tpu as pltpu

N = 10000
D = 128
H = 32
C = 40
EPS = 1e-5
BM = 200
SCALE = 1.0 / math.sqrt(1.0 + EPS)


def _pass1_kernel(adj_ref, x_ref, wcat_ref, s0_ref, s10_ref,
                  v0a_ref, v0b_ref, v10a_ref, v10b_ref,
                  wca_ref, wcb_ref, bc_ref,
                  sub2a_ref, acc_ref, scat_ref):
    i = pl.program_id(0)

    @pl.when(i == 0)
    def _():
        scat_ref[...] = jnp.dot(x_ref[...], wcat_ref[...],
                                preferred_element_type=jnp.float32)

    x_blk = x_ref[pl.ds(i * BM, BM), :]
    t = jnp.dot(adj_ref[...], scat_ref[...],
                preferred_element_type=jnp.float32)

    s1 = (t[:, :H] + jnp.dot(x_blk, s0_ref[...],
                             preferred_element_type=jnp.float32))
    s1 = jnp.maximum(s1 * v0a_ref[...] + v0b_ref[...], 0.0)

    s2a = (t[:, H:] + jnp.dot(x_blk, s10_ref[...],
                              preferred_element_type=jnp.float32))
    s2a = jnp.maximum(s2a * v10a_ref[...] + v10b_ref[...], 0.0)

    sub2a_ref[...] = s2a
    acc_ref[...] = (jnp.dot(x_blk, wca_ref[...],
                            preferred_element_type=jnp.float32)
                    + jnp.dot(s1, wcb_ref[...],
                              preferred_element_type=jnp.float32)
                    + bc_ref[...])


def _pass2_kernel(adj_ref, a_ref, w11_ref, s11_ref,
                  v11a_ref, v11b_ref, wcc_ref, acc_ref,
                  out_ref, sup_ref):
    i = pl.program_id(0)

    @pl.when(i == 0)
    def _():
        sup_ref[...] = jnp.dot(a_ref[...], w11_ref[...],
                               preferred_element_type=jnp.float32)

    a_blk = a_ref[pl.ds(i * BM, BM), :]
    t = jnp.dot(adj_ref[...], sup_ref[...],
                preferred_element_type=jnp.float32)
    s2 = t + jnp.dot(a_blk, s11_ref[...], preferred_element_type=jnp.float32)
    s2 = jnp.maximum(s2 * v11a_ref[...] + v11b_ref[...], 0.0)
    out_ref[...] = acc_ref[...] + jnp.dot(
        s2, wcc_ref[...], preferred_element_type=jnp.float32)


def _const_spec(shape):
    return pl.BlockSpec(shape, lambda i: (0,) * len(shape))


@jax.jit
def kernel(input, adj, W0, S0, b0, g0, be0, W10, S10, b10, g10, be10,
           W11, S11, b11, g11, be11, Wc, bc):
    x = input
    grid = (N // BM,)

    def fold(b, g, be):
        va = (SCALE * g).reshape(1, H)
        vb = (b * SCALE * g + be).reshape(1, H)
        return va, vb

    v0a, v0b = fold(b0, g0, be0)
    v10a, v10b = fold(b10, g10, be10)
    v11a, v11b = fold(b11, g11, be11)

    wcat = jnp.concatenate([W0, W10], axis=1)
    wca = Wc[:D]
    wcb = Wc[D:D + H]
    wcc = Wc[D + H:]
    bc2 = bc.reshape(1, C)

    sub2a, acc = pl.pallas_call(
        _pass1_kernel,
        grid=grid,
        in_specs=[
            pl.BlockSpec((BM, N), lambda i: (i, 0)),
            _const_spec((N, D)),
            _const_spec((D, 2 * H)),
            _const_spec((D, H)),
            _const_spec((D, H)),
            _const_spec((1, H)), _const_spec((1, H)),
            _const_spec((1, H)), _const_spec((1, H)),
            _const_spec((D, C)),
            _const_spec((H, C)),
            _const_spec((1, C)),
        ],
        out_specs=[
            pl.BlockSpec((BM, H), lambda i: (i, 0)),
            pl.BlockSpec((BM, C), lambda i: (i, 0)),
        ],
        out_shape=[
            jax.ShapeDtypeStruct((N, H), jnp.float32),
            jax.ShapeDtypeStruct((N, C), jnp.float32),
        ],
        scratch_shapes=[pltpu.VMEM((N, 2 * H), jnp.float32)],
    )(adj, x, wcat, S0, S10, v0a, v0b, v10a, v10b, wca, wcb, bc2)

    out = pl.pallas_call(
        _pass2_kernel,
        grid=grid,
        in_specs=[
            pl.BlockSpec((BM, N), lambda i: (i, 0)),
            _const_spec((N, H)),
            _const_spec((H, H)),
            _const_spec((H, H)),
            _const_spec((1, H)), _const_spec((1, H)),
            _const_spec((H, C)),
            pl.BlockSpec((BM, C), lambda i: (i, 0)),
        ],
        out_specs=pl.BlockSpec((BM, C), lambda i: (i, 0)),
        out_shape=jax.ShapeDtypeStruct((N, C), jnp.float32),
        scratch_shapes=[pltpu.VMEM((N, H), jnp.float32)],
    )(adj, sub2a, W11, S11, v11a, v11b, wcc, acc)

    return out


# speedup vs baseline: 1.0740x; 1.0740x over previous
import math

import jax
import jax.numpy as jnp
from jax.experimental import pallas as pl
from jax.experimental.pallas import tpu as pltpu

N = 10000
D = 128
H = 32
C = 40
EPS = 1e-5
BM = 400
NBLK = N // BM
SCALE = 1.0 / math.sqrt(1.0 + EPS)


def _fused_kernel(adj_ref, x_ref, wcat_ref, s0_ref, s10_ref,
                  v0a_ref, v0b_ref, v10a_ref, v10b_ref,
                  wca_ref, wcb_ref, bc_ref,
                  w11_ref, s11w_ref, v11a_ref, v11b_ref, wcc_ref,
                  out_ref, scat_ref, s11_ref, a_ref, acc_ref):
    i = pl.program_id(0)
    j = jax.lax.rem(i, NBLK)
    row = j * BM

    @pl.when(i == 0)
    def _():
        scat_ref[...] = jnp.dot(x_ref[...], wcat_ref[...],
                                preferred_element_type=jnp.float32)

    @pl.when(i < NBLK)
    def _():
        x_blk = x_ref[pl.ds(row, BM), :]
        t = jnp.dot(adj_ref[...], scat_ref[...],
                    preferred_element_type=jnp.float32)
        s1 = t[:, :H] + jnp.dot(x_blk, s0_ref[...],
                                preferred_element_type=jnp.float32)
        s1 = jnp.maximum(s1 * v0a_ref[...] + v0b_ref[...], 0.0)
        s2a = t[:, H:] + jnp.dot(x_blk, s10_ref[...],
                                 preferred_element_type=jnp.float32)
        s2a = jnp.maximum(s2a * v10a_ref[...] + v10b_ref[...], 0.0)
        a_ref[pl.ds(row, BM), :] = s2a
        acc_ref[pl.ds(row, BM), :] = (
            jnp.dot(x_blk, wca_ref[...], preferred_element_type=jnp.float32)
            + jnp.dot(s1, wcb_ref[...], preferred_element_type=jnp.float32)
            + bc_ref[...])

    @pl.when(i == NBLK)
    def _():
        s11_ref[...] = jnp.dot(a_ref[...], w11_ref[...],
                               preferred_element_type=jnp.float32)

    @pl.when(i >= NBLK)
    def _():
        a_blk = a_ref[pl.ds(row, BM), :]
        t = jnp.dot(adj_ref[...], s11_ref[...],
                    preferred_element_type=jnp.float32)
        s2 = t + jnp.dot(a_blk, s11w_ref[...],
                         preferred_element_type=jnp.float32)
        s2 = jnp.maximum(s2 * v11a_ref[...] + v11b_ref[...], 0.0)
        out_ref[...] = acc_ref[pl.ds(row, BM), :] + jnp.dot(
            s2, wcc_ref[...], preferred_element_type=jnp.float32)


def _const_spec(shape):
    return pl.BlockSpec(shape, lambda i: (0,) * len(shape))


@jax.jit
def kernel(input, adj, W0, S0, b0, g0, be0, W10, S10, b10, g10, be10,
           W11, S11, b11, g11, be11, Wc, bc):
    x = input

    def fold(b, g, be):
        va = (SCALE * g).reshape(1, H)
        vb = (b * SCALE * g + be).reshape(1, H)
        return va, vb

    v0a, v0b = fold(b0, g0, be0)
    v10a, v10b = fold(b10, g10, be10)
    v11a, v11b = fold(b11, g11, be11)

    wcat = jnp.concatenate([W0, W10], axis=1)
    wca = Wc[:D]
    wcb = Wc[D:D + H]
    wcc = Wc[D + H:]
    bc2 = bc.reshape(1, C)

    out = pl.pallas_call(
        _fused_kernel,
        grid=(2 * NBLK,),
        in_specs=[
            pl.BlockSpec((BM, N), lambda i: (jax.lax.rem(i, NBLK), 0)),
            _const_spec((N, D)),
            _const_spec((D, 2 * H)),
            _const_spec((D, H)),
            _const_spec((D, H)),
            _const_spec((1, H)), _const_spec((1, H)),
            _const_spec((1, H)), _const_spec((1, H)),
            _const_spec((D, C)),
            _const_spec((H, C)),
            _const_spec((1, C)),
            _const_spec((H, H)),
            _const_spec((H, H)),
            _const_spec((1, H)), _const_spec((1, H)),
            _const_spec((H, C)),
        ],
        out_specs=pl.BlockSpec(
            (BM, C), lambda i: (jnp.maximum(i - NBLK, 0), 0)),
        out_shape=jax.ShapeDtypeStruct((N, C), jnp.float32),
        scratch_shapes=[
            pltpu.VMEM((N, 2 * H), jnp.float32),
            pltpu.VMEM((N, H), jnp.float32),
            pltpu.VMEM((N, H), jnp.float32),
            pltpu.VMEM((N, C), jnp.float32),
        ],
    )(adj, x, wcat, S0, S10, v0a, v0b, v10a, v10b, wca, wcb, bc2,
      W11, S11, v11a, v11b, wcc)

    return out


# speedup vs baseline: 1.0877x; 1.0128x over previous
import math

import jax
import jax.numpy as jnp
from jax.experimental import pallas as pl
from jax.experimental.pallas import tpu as pltpu

N = 10000
D = 128
H = 32
C = 40
EPS = 1e-5
BM = 400
NBLK = N // BM
SCALE = 1.0 / math.sqrt(1.0 + EPS)


def _row_block(i):
    return jnp.where(i < NBLK, i, 2 * NBLK - 1 - i)


def _fused_kernel(adj_ref, x_ref, wcat_ref, s0_ref, s10_ref,
                  v0a_ref, v0b_ref, v10a_ref, v10b_ref,
                  wca_ref, wcb_ref, bc_ref,
                  w11_ref, s11w_ref, v11a_ref, v11b_ref, wcc_ref,
                  out_ref, scat_ref, s11_ref, a_ref, acc_ref):
    i = pl.program_id(0)
    row = _row_block(i) * BM

    @pl.when(i == 0)
    def _():
        scat_ref[...] = jnp.dot(x_ref[...], wcat_ref[...],
                                preferred_element_type=jnp.float32)

    @pl.when(i < NBLK)
    def _():
        x_blk = x_ref[pl.ds(row, BM), :]
        t = jnp.dot(adj_ref[...], scat_ref[...],
                    preferred_element_type=jnp.float32)
        s1 = t[:, :H] + jnp.dot(x_blk, s0_ref[...],
                                preferred_element_type=jnp.float32)
        s1 = jnp.maximum(s1 * v0a_ref[...] + v0b_ref[...], 0.0)
        s2a = t[:, H:] + jnp.dot(x_blk, s10_ref[...],
                                 preferred_element_type=jnp.float32)
        s2a = jnp.maximum(s2a * v10a_ref[...] + v10b_ref[...], 0.0)
        a_ref[pl.ds(row, BM), :] = s2a
        s11_ref[pl.ds(row, BM), :] = jnp.dot(
            s2a, w11_ref[...], preferred_element_type=jnp.float32)
        acc_ref[pl.ds(row, BM), :] = (
            jnp.dot(x_blk, wca_ref[...], preferred_element_type=jnp.float32)
            + jnp.dot(s1, wcb_ref[...], preferred_element_type=jnp.float32)
            + bc_ref[...])

    @pl.when(i >= NBLK)
    def _():
        a_blk = a_ref[pl.ds(row, BM), :]
        t = jnp.dot(adj_ref[...], s11_ref[...],
                    preferred_element_type=jnp.float32)
        s2 = t + jnp.dot(a_blk, s11w_ref[...],
                         preferred_element_type=jnp.float32)
        s2 = jnp.maximum(s2 * v11a_ref[...] + v11b_ref[...], 0.0)
        out_ref[...] = acc_ref[pl.ds(row, BM), :] + jnp.dot(
            s2, wcc_ref[...], preferred_element_type=jnp.float32)


def _const_spec(shape):
    return pl.BlockSpec(shape, lambda i: (0,) * len(shape))


@jax.jit
def kernel(input, adj, W0, S0, b0, g0, be0, W10, S10, b10, g10, be10,
           W11, S11, b11, g11, be11, Wc, bc):
    x = input

    def fold(b, g, be):
        va = (SCALE * g).reshape(1, H)
        vb = (b * SCALE * g + be).reshape(1, H)
        return va, vb

    v0a, v0b = fold(b0, g0, be0)
    v10a, v10b = fold(b10, g10, be10)
    v11a, v11b = fold(b11, g11, be11)

    wcat = jnp.concatenate([W0, W10], axis=1)
    wca = Wc[:D]
    wcb = Wc[D:D + H]
    wcc = Wc[D + H:]
    bc2 = bc.reshape(1, C)

    out = pl.pallas_call(
        _fused_kernel,
        grid=(2 * NBLK,),
        in_specs=[
            pl.BlockSpec((BM, N), lambda i: (_row_block(i), 0)),
            _const_spec((N, D)),
            _const_spec((D, 2 * H)),
            _const_spec((D, H)),
            _const_spec((D, H)),
            _const_spec((1, H)), _const_spec((1, H)),
            _const_spec((1, H)), _const_spec((1, H)),
            _const_spec((D, C)),
            _const_spec((H, C)),
            _const_spec((1, C)),
            _const_spec((H, H)),
            _const_spec((H, H)),
            _const_spec((1, H)), _const_spec((1, H)),
            _const_spec((H, C)),
        ],
        out_specs=pl.BlockSpec(
            (BM, C), lambda i: (jnp.where(i < NBLK, NBLK - 1, 2 * NBLK - 1 - i), 0)),
        out_shape=jax.ShapeDtypeStruct((N, C), jnp.float32),
        scratch_shapes=[
            pltpu.VMEM((N, 2 * H), jnp.float32),
            pltpu.VMEM((N, H), jnp.float32),
            pltpu.VMEM((N, H), jnp.float32),
            pltpu.VMEM((N, C), jnp.float32),
        ],
    )(adj, x, wcat, S0, S10, v0a, v0b, v10a, v10b, wca, wcb, bc2,
      W11, S11, v11a, v11b, wcc)

    return out
